# Initial kernel scaffold; baseline (speedup 1.0000x reference)
import jax
import jax.numpy as jnp
from jax.experimental import pallas as pl


def kernel(x, edge_index, W1, b1, W2, b2, Wfc, bfc):
    raise NotImplementedError("write your pallas kernel here")


# speedup vs baseline: 21.7164x; 21.7164x over previous
import functools

import jax
import jax.numpy as jnp
from jax import lax
from jax.experimental import pallas as pl
from jax.experimental.pallas import tpu as pltpu
from jax.experimental.pallas import tpu_sc as plsc

N_NODES = 10000
D = 128
N_P = 10240
NC, NS = 2, 16
NW = NC * NS
ROWS_PER_TILE = N_P // NS
CHUNK = 128
CHUNKS = 80
EDGES_P = NW * CHUNKS * CHUNK
ROW_BLK = 1024
GRID = N_P // ROW_BLK

def _sc_mesh():
    return plsc.VectorSubcoreMesh(
        core_axis_name="c", subcore_axis_name="s", num_cores=NC, num_subcores=NS)


def _deg_body(dst_hbm, zeros1_hbm, out_hbm, didx, ones_v, hist):
    cid = lax.axis_index("c")
    sid = lax.axis_index("s")
    wid = sid * NC + cid
    row0 = sid * ROWS_PER_TILE
    pltpu.sync_copy(zeros1_hbm.at[pl.ds(row0, ROWS_PER_TILE)],
                    hist.at[pl.ds(row0, ROWS_PER_TILE)])
    pltpu.sync_copy(dst_hbm.at[wid], didx)
    for i in range(CHUNK // 16):
        ones_v[pl.ds(i * 16, 16)] = jnp.ones((16,), jnp.float32)
    plsc.subcore_barrier()

    def body(j, carry):
        pltpu.sync_copy(ones_v, hist.at[didx.at[j]], add=True)
        return carry

    lax.fori_loop(0, CHUNKS, body, 0)
    plsc.subcore_barrier()
    pltpu.sync_copy(hist.at[pl.ds(row0, ROWS_PER_TILE)],
                    out_hbm.at[cid].at[pl.ds(row0, ROWS_PER_TILE)])


@functools.cache
def _deg_call():
    return pl.kernel(
        _deg_body,
        out_type=jax.ShapeDtypeStruct((NC, N_P), jnp.float32),
        mesh=_sc_mesh(),
        scratch_types=[
            pltpu.VMEM((CHUNKS, CHUNK), jnp.int32),
            pltpu.VMEM((CHUNK,), jnp.float32),
            pltpu.VMEM_SHARED((N_P,), jnp.float32),
        ],
    )


def _agg_body(m_hbm, src_hbm, dst_hbm, zeros2_hbm, out_hbm,
              sidx, didx, gbuf, acc, gsem):
    cid = lax.axis_index("c")
    sid = lax.axis_index("s")
    wid = sid * NC + cid
    row0 = sid * ROWS_PER_TILE
    pltpu.sync_copy(zeros2_hbm.at[pl.ds(row0, ROWS_PER_TILE)],
                    acc.at[pl.ds(row0, ROWS_PER_TILE)])
    pltpu.sync_copy(src_hbm.at[wid], sidx)
    pltpu.sync_copy(dst_hbm.at[wid], didx)
    plsc.subcore_barrier()

    def body(j, carry):
        pltpu.async_copy(m_hbm.at[sidx.at[j]], gbuf, gsem).wait()
        pltpu.sync_copy(gbuf, acc.at[didx.at[j]], add=True)
        return carry

    lax.fori_loop(0, CHUNKS, body, 0)
    plsc.subcore_barrier()
    pltpu.sync_copy(acc.at[pl.ds(row0, ROWS_PER_TILE)],
                    out_hbm.at[cid].at[pl.ds(row0, ROWS_PER_TILE)])


@functools.cache
def _agg_call():
    return pl.kernel(
        _agg_body,
        out_type=jax.ShapeDtypeStruct((NC, N_P, D), jnp.float32),
        mesh=_sc_mesh(),
        scratch_types=[
            pltpu.VMEM((CHUNKS, CHUNK), jnp.int32),
            pltpu.VMEM((CHUNKS, CHUNK), jnp.int32),
            pltpu.VMEM((CHUNK, D), jnp.float32),
            pltpu.VMEM_SHARED((N_P, D), jnp.float32),
            pltpu.SemaphoreType.DMA,
        ],
    )


def _dinv_bcast(deg0, deg1):
    deg = deg0 + deg1 + 1.0
    dinv = lax.rsqrt(deg)
    a = jnp.broadcast_to(dinv[None, :], (D, dinv.shape[0]))
    b = jnp.full((D, D), 1.0 / D, jnp.float32)
    return lax.dot_general(a, b, (((0,), (0,)), ((), ())),
                           preferred_element_type=jnp.float32)


def _tc1_body(deg0_ref, deg1_ref, x_ref, w_ref, m_ref, dinv_ref):
    dinvb = _dinv_bcast(deg0_ref[...], deg1_ref[...])
    h = jnp.dot(x_ref[...], w_ref[...], preferred_element_type=jnp.float32)
    dinv_ref[...] = dinvb
    m_ref[...] = dinvb * h


def _tc2_body(p0_ref, p1_ref, m_ref, dinv_ref, b_ref, w_ref, out_ref):
    s = p0_ref[...] + p1_ref[...] + m_ref[...]
    a = jnp.maximum(dinv_ref[...] * s + b_ref[...], 0.0)
    h = jnp.dot(a, w_ref[...], preferred_element_type=jnp.float32)
    out_ref[...] = dinv_ref[...] * h


def _tc3_body(p0_ref, p1_ref, m_ref, dinv_ref, b_ref, w_ref, bfc_ref, out_ref):
    s = p0_ref[...] + p1_ref[...] + m_ref[...]
    a = jnp.maximum(dinv_ref[...] * s + b_ref[...], 0.0)
    out_ref[...] = jnp.dot(a, w_ref[...],
                           preferred_element_type=jnp.float32) + bfc_ref[...]


_row_spec = pl.BlockSpec((ROW_BLK, D), lambda i: (i, 0))
_vec_spec = pl.BlockSpec((ROW_BLK,), lambda i: (i,))
_w_spec = pl.BlockSpec((D, D), lambda i: (0, 0))
_b_spec = pl.BlockSpec((1, D), lambda i: (0, 0))

_tc1_call = pl.pallas_call(
    _tc1_body,
    grid=(GRID,),
    in_specs=[_vec_spec, _vec_spec, _row_spec, _w_spec],
    out_specs=[_row_spec, _row_spec],
    out_shape=[jax.ShapeDtypeStruct((N_P, D), jnp.float32),
               jax.ShapeDtypeStruct((N_P, D), jnp.float32)],
)

_tc2_call = pl.pallas_call(
    _tc2_body,
    grid=(GRID,),
    in_specs=[_row_spec, _row_spec, _row_spec, _row_spec, _b_spec, _w_spec],
    out_specs=_row_spec,
    out_shape=jax.ShapeDtypeStruct((N_P, D), jnp.float32),
)

_tc3_call = pl.pallas_call(
    _tc3_body,
    grid=(GRID,),
    in_specs=[_row_spec, _row_spec, _row_spec, _row_spec, _b_spec, _w_spec,
              _b_spec],
    out_specs=_row_spec,
    out_shape=jax.ShapeDtypeStruct((N_P, D), jnp.float32),
)


def kernel(x, edge_index, W1, b1, W2, b2, Wfc, bfc):
    e = jnp.asarray(edge_index, jnp.int32)
    n_pad = EDGES_P - e.shape[1]
    k = jnp.arange(n_pad, dtype=jnp.int32)
    src_p = jnp.concatenate([e[0], k % N_NODES]).reshape(NW, CHUNKS, CHUNK)
    dst_p = jnp.concatenate([e[1], N_NODES + k % (N_P - N_NODES)]
                            ).reshape(NW, CHUNKS, CHUNK)

    x_p = jnp.pad(x, ((0, N_P - x.shape[0]), (0, 0)))
    zeros1 = jnp.zeros((N_P,), jnp.float32)
    zeros2 = jnp.zeros((N_P, D), jnp.float32)
    b1r = b1.reshape(1, D)
    b2r = b2.reshape(1, D)
    bfcr = bfc.reshape(1, D)

    deg = _deg_call()(dst_p, zeros1)
    m1, dinvb = _tc1_call(deg[0], deg[1], x_p, W1)
    p1 = _agg_call()(m1, src_p, dst_p, zeros2)
    m2 = _tc2_call(p1[0], p1[1], m1, dinvb, b1r, W2)
    p2 = _agg_call()(m2, src_p, dst_p, zeros2)
    out = _tc3_call(p2[0], p2[1], m2, dinvb, b2r, Wfc, bfcr)
    return out[:N_NODES]


# speedup vs baseline: 28.3503x; 1.3055x over previous
import functools

import jax
import jax.numpy as jnp
from jax import lax
from jax.experimental import pallas as pl
from jax.experimental.pallas import tpu as pltpu
from jax.experimental.pallas import tpu_sc as plsc

N_NODES = 10000
D = 128
N_P = 10240
NC, NS = 2, 16
NW = NC * NS
ROWS_PER_TILE = N_P // NS
CHUNK = 128
CHUNKS = 80
EDGES_P = NW * CHUNKS * CHUNK
ROW_BLK = 1024
GRID = N_P // ROW_BLK

def _sc_mesh():
    return plsc.VectorSubcoreMesh(
        core_axis_name="c", subcore_axis_name="s", num_cores=NC, num_subcores=NS)


def _deg_body(dst_hbm, zeros1_hbm, out_hbm, didx, ones_v, hist):
    cid = lax.axis_index("c")
    sid = lax.axis_index("s")
    wid = sid * NC + cid
    row0 = sid * ROWS_PER_TILE
    pltpu.sync_copy(zeros1_hbm.at[pl.ds(row0, ROWS_PER_TILE)],
                    hist.at[pl.ds(row0, ROWS_PER_TILE)])
    pltpu.sync_copy(dst_hbm.at[wid], didx)
    for i in range(CHUNK // 16):
        ones_v[pl.ds(i * 16, 16)] = jnp.ones((16,), jnp.float32)
    plsc.subcore_barrier()

    def body(j, carry):
        pltpu.sync_copy(ones_v, hist.at[didx.at[j]], add=True)
        return carry

    lax.fori_loop(0, CHUNKS, body, 0)
    plsc.subcore_barrier()
    pltpu.sync_copy(hist.at[pl.ds(row0, ROWS_PER_TILE)],
                    out_hbm.at[cid].at[pl.ds(row0, ROWS_PER_TILE)])


@functools.cache
def _deg_call():
    return pl.kernel(
        _deg_body,
        out_type=jax.ShapeDtypeStruct((NC, N_P), jnp.float32),
        mesh=_sc_mesh(),
        scratch_types=[
            pltpu.VMEM((CHUNKS, CHUNK), jnp.int32),
            pltpu.VMEM((CHUNK,), jnp.float32),
            pltpu.VMEM_SHARED((N_P,), jnp.float32),
        ],
    )


def _agg_body(m_hbm, src_hbm, dst_hbm, zeros2_hbm, out_hbm,
              srcb, dstb, gbuf0, gbuf1, acc, isem0, isem1, gsem0, gsem1):
    cid = lax.axis_index("c")
    sid = lax.axis_index("s")
    wid = sid * NC + cid
    row0 = sid * ROWS_PER_TILE
    pltpu.sync_copy(zeros2_hbm.at[pl.ds(row0, ROWS_PER_TILE)],
                    acc.at[pl.ds(row0, ROWS_PER_TILE)])

    def fire_idx(j, slot, sem):
        pltpu.async_copy(src_hbm.at[wid].at[j], srcb.at[slot], sem)
        pltpu.async_copy(dst_hbm.at[wid].at[j], dstb.at[slot], sem)

    def wait_idx(slot, sem):
        pltpu.make_async_copy(src_hbm.at[wid].at[0], srcb.at[slot], sem).wait()
        pltpu.make_async_copy(dst_hbm.at[wid].at[0], dstb.at[slot], sem).wait()

    fire_idx(0, 0, isem0)
    fire_idx(1, 1, isem1)
    plsc.subcore_barrier()
    wait_idx(0, isem0)
    pltpu.async_copy(m_hbm.at[srcb.at[0]], gbuf0, gsem0)

    def body(jj, carry):
        j1 = 2 * jj + 1
        j2 = jnp.where(j1 + 1 < CHUNKS, j1 + 1, 0)
        j3 = jnp.where(j1 + 2 < CHUNKS, j1 + 2, 1)
        wait_idx(1, isem1)
        pltpu.async_copy(m_hbm.at[srcb.at[1]], gbuf1, gsem1)
        pltpu.make_async_copy(m_hbm.at[srcb.at[0]], gbuf0, gsem0).wait()
        pltpu.sync_copy(gbuf0, acc.at[dstb.at[0]], add=True)
        fire_idx(j2, 0, isem0)
        wait_idx(0, isem0)
        pltpu.async_copy(m_hbm.at[srcb.at[0]], gbuf0, gsem0)
        pltpu.make_async_copy(m_hbm.at[srcb.at[1]], gbuf1, gsem1).wait()
        pltpu.sync_copy(gbuf1, acc.at[dstb.at[1]], add=True)
        fire_idx(j3, 1, isem1)
        return carry

    lax.fori_loop(0, CHUNKS // 2, body, 0)
    wait_idx(1, isem1)
    pltpu.make_async_copy(m_hbm.at[srcb.at[0]], gbuf0, gsem0).wait()
    plsc.subcore_barrier()
    pltpu.sync_copy(acc.at[pl.ds(row0, ROWS_PER_TILE)],
                    out_hbm.at[cid].at[pl.ds(row0, ROWS_PER_TILE)])


@functools.cache
def _agg_call():
    return pl.kernel(
        _agg_body,
        out_type=jax.ShapeDtypeStruct((NC, N_P, D), jnp.float32),
        mesh=_sc_mesh(),
        scratch_types=[
            pltpu.VMEM((2, CHUNK), jnp.int32),
            pltpu.VMEM((2, CHUNK), jnp.int32),
            pltpu.VMEM((CHUNK, D), jnp.float32),
            pltpu.VMEM((CHUNK, D), jnp.float32),
            pltpu.VMEM_SHARED((N_P, D), jnp.float32),
            pltpu.SemaphoreType.DMA,
            pltpu.SemaphoreType.DMA,
            pltpu.SemaphoreType.DMA,
            pltpu.SemaphoreType.DMA,
        ],
    )


def _dinv_bcast(deg0, deg1):
    deg = deg0 + deg1 + 1.0
    dinv = lax.rsqrt(deg)
    a = jnp.broadcast_to(dinv[None, :], (D, dinv.shape[0]))
    b = jnp.full((D, D), 1.0 / D, jnp.float32)
    return lax.dot_general(a, b, (((0,), (0,)), ((), ())),
                           preferred_element_type=jnp.float32)


def _tc1_body(deg0_ref, deg1_ref, x_ref, w_ref, m_ref, dinv_ref):
    dinvb = _dinv_bcast(deg0_ref[...], deg1_ref[...])
    h = jnp.dot(x_ref[...], w_ref[...], preferred_element_type=jnp.float32)
    dinv_ref[...] = dinvb
    m_ref[...] = dinvb * h


def _tc2_body(p0_ref, p1_ref, m_ref, dinv_ref, b_ref, w_ref, out_ref):
    s = p0_ref[...] + p1_ref[...] + m_ref[...]
    a = jnp.maximum(dinv_ref[...] * s + b_ref[...], 0.0)
    h = jnp.dot(a, w_ref[...], preferred_element_type=jnp.float32)
    out_ref[...] = dinv_ref[...] * h


def _tc3_body(p0_ref, p1_ref, m_ref, dinv_ref, b_ref, w_ref, bfc_ref, out_ref):
    s = p0_ref[...] + p1_ref[...] + m_ref[...]
    a = jnp.maximum(dinv_ref[...] * s + b_ref[...], 0.0)
    out_ref[...] = jnp.dot(a, w_ref[...],
                           preferred_element_type=jnp.float32) + bfc_ref[...]


_row_spec = pl.BlockSpec((ROW_BLK, D), lambda i: (i, 0))
_vec_spec = pl.BlockSpec((ROW_BLK,), lambda i: (i,))
_w_spec = pl.BlockSpec((D, D), lambda i: (0, 0))
_b_spec = pl.BlockSpec((1, D), lambda i: (0, 0))

_tc1_call = pl.pallas_call(
    _tc1_body,
    grid=(GRID,),
    in_specs=[_vec_spec, _vec_spec, _row_spec, _w_spec],
    out_specs=[_row_spec, _row_spec],
    out_shape=[jax.ShapeDtypeStruct((N_P, D), jnp.float32),
               jax.ShapeDtypeStruct((N_P, D), jnp.float32)],
)

_tc2_call = pl.pallas_call(
    _tc2_body,
    grid=(GRID,),
    in_specs=[_row_spec, _row_spec, _row_spec, _row_spec, _b_spec, _w_spec],
    out_specs=_row_spec,
    out_shape=jax.ShapeDtypeStruct((N_P, D), jnp.float32),
)

_tc3_call = pl.pallas_call(
    _tc3_body,
    grid=(GRID,),
    in_specs=[_row_spec, _row_spec, _row_spec, _row_spec, _b_spec, _w_spec,
              _b_spec],
    out_specs=_row_spec,
    out_shape=jax.ShapeDtypeStruct((N_P, D), jnp.float32),
)


def kernel(x, edge_index, W1, b1, W2, b2, Wfc, bfc):
    e = jnp.asarray(edge_index, jnp.int32)
    n_pad = EDGES_P - e.shape[1]
    k = jnp.arange(n_pad, dtype=jnp.int32)
    src_p = jnp.concatenate([e[0], k % N_NODES]).reshape(NW, CHUNKS, CHUNK)
    dst_p = jnp.concatenate([e[1], N_NODES + k % (N_P - N_NODES)]
                            ).reshape(NW, CHUNKS, CHUNK)

    x_p = jnp.pad(x, ((0, N_P - x.shape[0]), (0, 0)))
    zeros1 = jnp.zeros((N_P,), jnp.float32)
    zeros2 = jnp.zeros((N_P, D), jnp.float32)
    b1r = b1.reshape(1, D)
    b2r = b2.reshape(1, D)
    bfcr = bfc.reshape(1, D)

    deg = _deg_call()(dst_p, zeros1)
    m1, dinvb = _tc1_call(deg[0], deg[1], x_p, W1)
    p1 = _agg_call()(m1, src_p, dst_p, zeros2)
    m2 = _tc2_call(p1[0], p1[1], m1, dinvb, b1r, W2)
    p2 = _agg_call()(m2, src_p, dst_p, zeros2)
    out = _tc3_call(p2[0], p2[1], m2, dinvb, b2r, Wfc, bfcr)
    return out[:N_NODES]


# speedup vs baseline: 30.0155x; 1.0587x over previous
import functools

import jax
import jax.numpy as jnp
from jax import lax
from jax.experimental import pallas as pl
from jax.experimental.pallas import tpu as pltpu
from jax.experimental.pallas import tpu_sc as plsc

N_NODES = 10000
D = 128
N_P = 10240
NC, NS = 2, 16
NW = NC * NS
ROWS_PER_TILE = N_P // NS
CHUNK = 128
CHUNKS = 80
EDGES_P = NW * CHUNKS * CHUNK
ROW_BLK = 1024
GRID = N_P // ROW_BLK

def _sc_mesh():
    return plsc.VectorSubcoreMesh(
        core_axis_name="c", subcore_axis_name="s", num_cores=NC, num_subcores=NS)


def _deg_body(dst_hbm, zeros1_hbm, out_hbm, didx, ones_v, hist):
    cid = lax.axis_index("c")
    sid = lax.axis_index("s")
    wid = sid * NC + cid
    row0 = sid * ROWS_PER_TILE
    pltpu.sync_copy(zeros1_hbm.at[pl.ds(row0, ROWS_PER_TILE)],
                    hist.at[pl.ds(row0, ROWS_PER_TILE)])
    pltpu.sync_copy(dst_hbm.at[wid], didx)
    for i in range(CHUNK // 16):
        ones_v[pl.ds(i * 16, 16)] = jnp.ones((16,), jnp.float32)
    plsc.subcore_barrier()

    def body(j, carry):
        pltpu.sync_copy(ones_v, hist.at[didx.at[j]], add=True)
        return carry

    lax.fori_loop(0, CHUNKS, body, 0)
    plsc.subcore_barrier()
    pltpu.sync_copy(hist.at[pl.ds(row0, ROWS_PER_TILE)],
                    out_hbm.at[cid].at[pl.ds(row0, ROWS_PER_TILE)])


@functools.cache
def _deg_call():
    return pl.kernel(
        _deg_body,
        out_type=jax.ShapeDtypeStruct((NC, N_P), jnp.float32),
        mesh=_sc_mesh(),
        scratch_types=[
            pltpu.VMEM((CHUNKS, CHUNK), jnp.int32),
            pltpu.VMEM((CHUNK,), jnp.float32),
            pltpu.VMEM_SHARED((N_P,), jnp.float32),
        ],
    )


def _agg_body(m_hbm, src_hbm, dst_hbm, zeros2_hbm, out_hbm,
              srcb, dstb, gbuf0, gbuf1, acc, isem0, isem1, gsem0, gsem1):
    cid = lax.axis_index("c")
    sid = lax.axis_index("s")
    wid = sid * NC + cid
    row0 = sid * ROWS_PER_TILE
    pltpu.sync_copy(zeros2_hbm.at[pl.ds(row0, ROWS_PER_TILE)],
                    acc.at[pl.ds(row0, ROWS_PER_TILE)])

    def fire_idx(j, slot, sem):
        pltpu.async_copy(src_hbm.at[wid].at[j], srcb.at[slot], sem)
        pltpu.async_copy(dst_hbm.at[wid].at[j], dstb.at[slot], sem)

    def wait_idx(slot, sem):
        pltpu.make_async_copy(src_hbm.at[wid].at[0], srcb.at[slot], sem).wait()
        pltpu.make_async_copy(dst_hbm.at[wid].at[0], dstb.at[slot], sem).wait()

    fire_idx(0, 0, isem0)
    fire_idx(1, 1, isem1)
    plsc.subcore_barrier()
    wait_idx(0, isem0)
    pltpu.async_copy(m_hbm.at[srcb.at[0]], gbuf0, gsem0)

    def body(jj, carry):
        j1 = 2 * jj + 1
        j2 = jnp.where(j1 + 1 < CHUNKS, j1 + 1, 0)
        j3 = jnp.where(j1 + 2 < CHUNKS, j1 + 2, 1)
        wait_idx(1, isem1)
        pltpu.async_copy(m_hbm.at[srcb.at[1]], gbuf1, gsem1)
        pltpu.make_async_copy(m_hbm.at[srcb.at[0]], gbuf0, gsem0).wait()
        pltpu.sync_copy(gbuf0, acc.at[dstb.at[0]], add=True)
        fire_idx(j2, 0, isem0)
        wait_idx(0, isem0)
        pltpu.async_copy(m_hbm.at[srcb.at[0]], gbuf0, gsem0)
        pltpu.make_async_copy(m_hbm.at[srcb.at[1]], gbuf1, gsem1).wait()
        pltpu.sync_copy(gbuf1, acc.at[dstb.at[1]], add=True)
        fire_idx(j3, 1, isem1)
        return carry

    lax.fori_loop(0, CHUNKS // 2, body, 0)
    wait_idx(1, isem1)
    pltpu.make_async_copy(m_hbm.at[srcb.at[0]], gbuf0, gsem0).wait()
    plsc.subcore_barrier()
    pltpu.sync_copy(acc.at[pl.ds(row0, ROWS_PER_TILE)],
                    out_hbm.at[cid].at[pl.ds(row0, ROWS_PER_TILE)])


@functools.cache
def _agg_call():
    return pl.kernel(
        _agg_body,
        out_type=jax.ShapeDtypeStruct((NC, N_P, D), jnp.float32),
        mesh=_sc_mesh(),
        scratch_types=[
            pltpu.VMEM((2, CHUNK), jnp.int32),
            pltpu.VMEM((2, CHUNK), jnp.int32),
            pltpu.VMEM((CHUNK, D), jnp.float32),
            pltpu.VMEM((CHUNK, D), jnp.float32),
            pltpu.VMEM_SHARED((N_P, D), jnp.float32),
            pltpu.SemaphoreType.DMA,
            pltpu.SemaphoreType.DMA,
            pltpu.SemaphoreType.DMA,
            pltpu.SemaphoreType.DMA,
        ],
    )


def _dinv_bcast(deg0, deg1):
    deg = deg0 + deg1 + 1.0
    dinv = lax.rsqrt(deg)
    a = jnp.broadcast_to(dinv[None, :], (D, dinv.shape[0]))
    b = jnp.full((D, D), 1.0 / D, jnp.float32)
    return lax.dot_general(a, b, (((0,), (0,)), ((), ())),
                           preferred_element_type=jnp.float32)


def _tc1a_body(x_ref, w_ref, h_ref):
    h_ref[...] = jnp.dot(x_ref[...], w_ref[...],
                         preferred_element_type=jnp.float32)


def _tc1b_body(deg0_ref, deg1_ref, h_ref, m_ref, dinv_ref):
    dinvb = _dinv_bcast(deg0_ref[0, 0], deg1_ref[0, 0])
    dinv_ref[...] = dinvb
    m_ref[...] = dinvb * h_ref[...]


def _tc2_body(p0_ref, p1_ref, m_ref, dinv_ref, b_ref, w_ref, out_ref):
    s = p0_ref[0] + p1_ref[0] + m_ref[...]
    a = jnp.maximum(dinv_ref[...] * s + b_ref[...], 0.0)
    h = jnp.dot(a, w_ref[...], preferred_element_type=jnp.float32)
    out_ref[...] = dinv_ref[...] * h


def _tc3_body(p0_ref, p1_ref, m_ref, dinv_ref, b_ref, w_ref, bfc_ref, out_ref):
    s = p0_ref[0] + p1_ref[0] + m_ref[...]
    a = jnp.maximum(dinv_ref[...] * s + b_ref[...], 0.0)
    out_ref[...] = jnp.dot(a, w_ref[...],
                           preferred_element_type=jnp.float32) + bfc_ref[...]


_row_spec = pl.BlockSpec((ROW_BLK, D), lambda i: (i, 0))
_p0_spec = pl.BlockSpec((1, ROW_BLK, D), lambda i: (0, i, 0))
_p1_spec = pl.BlockSpec((1, ROW_BLK, D), lambda i: (1, i, 0))
_deg0_spec = pl.BlockSpec((1, 1, ROW_BLK), lambda i: (0, 0, i))
_deg1_spec = pl.BlockSpec((1, 1, ROW_BLK), lambda i: (1, 0, i))
_w_spec = pl.BlockSpec((D, D), lambda i: (0, 0))
_b_spec = pl.BlockSpec((1, D), lambda i: (0, 0))

_tc1a_call = pl.pallas_call(
    _tc1a_body,
    grid=(GRID,),
    in_specs=[_row_spec, _w_spec],
    out_specs=_row_spec,
    out_shape=jax.ShapeDtypeStruct((N_P, D), jnp.float32),
)

_tc1b_call = pl.pallas_call(
    _tc1b_body,
    grid=(GRID,),
    in_specs=[_deg0_spec, _deg1_spec, _row_spec],
    out_specs=[_row_spec, _row_spec],
    out_shape=[jax.ShapeDtypeStruct((N_P, D), jnp.float32),
               jax.ShapeDtypeStruct((N_P, D), jnp.float32)],
)

_tc2_call = pl.pallas_call(
    _tc2_body,
    grid=(GRID,),
    in_specs=[_p0_spec, _p1_spec, _row_spec, _row_spec, _b_spec, _w_spec],
    out_specs=_row_spec,
    out_shape=jax.ShapeDtypeStruct((N_P, D), jnp.float32),
)

_tc3_call = pl.pallas_call(
    _tc3_body,
    grid=(GRID,),
    in_specs=[_p0_spec, _p1_spec, _row_spec, _row_spec, _b_spec, _w_spec,
              _b_spec],
    out_specs=pl.BlockSpec((ROW_BLK, D), lambda i: (i, 0)),
    out_shape=jax.ShapeDtypeStruct((N_NODES, D), jnp.float32),
)


def kernel(x, edge_index, W1, b1, W2, b2, Wfc, bfc):
    e = jnp.asarray(edge_index, jnp.int32)
    n_pad = EDGES_P - e.shape[1]
    k = jnp.arange(n_pad, dtype=jnp.int32)
    src_p = jnp.concatenate([e[0], k % N_NODES]).reshape(NW, CHUNKS, CHUNK)
    dst_p = jnp.concatenate([e[1], N_NODES + k % (N_P - N_NODES)]
                            ).reshape(NW, CHUNKS, CHUNK)

    zeros1 = jnp.zeros((N_P,), jnp.float32)
    zeros2 = jnp.zeros((N_P, D), jnp.float32)
    b1r = b1.reshape(1, D)
    b2r = b2.reshape(1, D)
    bfcr = bfc.reshape(1, D)

    deg = _deg_call()(dst_p, zeros1).reshape(NC, 1, N_P)
    h1 = _tc1a_call(x, W1)
    m1, dinvb = _tc1b_call(deg, deg, h1)
    p1 = _agg_call()(m1, src_p, dst_p, zeros2)
    m2 = _tc2_call(p1, p1, m1, dinvb, b1r, W2)
    p2 = _agg_call()(m2, src_p, dst_p, zeros2)
    return _tc3_call(p2, p2, m2, dinvb, b2r, Wfc, bfcr)
